# Initial kernel scaffold; baseline (speedup 1.0000x reference)
import jax
import jax.numpy as jnp
from jax.experimental import pallas as pl


def kernel(brand, modelname, version, phone_log_model, phone_raw_model, total_use_days, user_age, user_sex, user_degree, resident_province, resident_city, resident_city_type, sale_channel_1, sale_channel_2, vatality, all_app, emb_brand, emb_model, emb_version, emb_phone_log, emb_phone_raw, phone_W1, phone_b1, phone_W2, phone_b2, emb_age, emb_sex, emb_degree, emb_province, emb_city, emb_city_type, emb_ch1, emb_ch2, emb_use_days, emb_vitality, conv_vit_k, conv_vit_b, conv_app_k, conv_app_b, user_W1, user_b1, user_W2, user_b2, rate_W1, rate_b1, rate_W2, rate_b2):
    raise NotImplementedError("write your pallas kernel here")


# speedup vs baseline: 1.0552x; 1.0552x over previous
import functools

import jax
import jax.numpy as jnp
from jax import lax
from jax.experimental import pallas as pl
from jax.experimental.pallas import tpu as pltpu
from jax.experimental.pallas import tpu_sc as plsc

B = 4096
BB = 128
NW = 32
BPW = B // NW

_SIZES = [100, 5000, 50, 10000, 10000, 100, 3, 10, 35, 400, 6, 50, 200]
_NT = len(_SIZES)


def _sc_gather(stacked, idx_all, days_tab, days_idx):
    info = plsc.get_sparse_core_info()
    nc = info.num_cores
    mesh = plsc.VectorSubcoreMesh(core_axis_name="c", subcore_axis_name="s")

    @functools.partial(
        pl.kernel,
        mesh=mesh,
        out_type=[
            jax.ShapeDtypeStruct((_NT, B, 16), jnp.float32),
            jax.ShapeDtypeStruct((B, 112), jnp.float32),
        ],
        scratch_types=[
            pltpu.VMEM((BPW,), jnp.int32),
            pltpu.VMEM((BPW, 16), jnp.float32),
            pltpu.VMEM((BPW, 112), jnp.float32),
            pltpu.SemaphoreType.DMA,
        ],
        compiler_params=pltpu.CompilerParams(use_tc_tiling_on_sc=False),
    )
    def gk(stacked_hbm, idx_hbm, days_hbm, didx_hbm, out1, out2,
           idx_v, rows_v, rows2_v, sem):
        wid = lax.axis_index("s") * nc + lax.axis_index("c")
        base = wid * BPW
        for t in range(_NT):
            pltpu.sync_copy(idx_hbm.at[pl.ds(t * B + base, BPW)], idx_v)
            pltpu.async_copy(stacked_hbm.at[idx_v], rows_v, sem).wait()
            pltpu.sync_copy(rows_v, out1.at[t, pl.ds(base, BPW)])
        pltpu.sync_copy(didx_hbm.at[pl.ds(base, BPW)], idx_v)
        pltpu.async_copy(days_hbm.at[idx_v], rows2_v, sem).wait()
        pltpu.sync_copy(rows2_v, out2.at[pl.ds(base, BPW)])

    return gk(stacked, idx_all, days_tab, days_idx)


def _dot(a, b):
    return jnp.dot(a, b, preferred_element_type=jnp.float32)


def _tc_body(gs_ref, gd_ref, x_ref, v_ref,
             pw1_ref, pb1_ref, pw2_ref, pb2_ref,
             uws_ref, uwd_ref, uwa_ref, uwv_ref, ub1_ref, uw2_ref, ub2_ref,
             kcat_ref, ab_ref, evit_ref, vitk_ref, vb_ref,
             rw1_ref, rb1_ref, rw2_ref, rb2_ref, out_ref):
    relu = jax.nn.relu
    u = ub1_ref[...] + _dot(gd_ref[...], uwd_ref[...])
    for i in range(8):
        u = u + _dot(gs_ref[5 + i], uws_ref[i])

    ab = ab_ref[...]
    kcat = kcat_ref[...]
    y0 = y1 = y2 = None
    c_prev = None
    for w in range(30):
        y0, y1, y2 = y1, y2, _dot(x_ref[:, w, :], kcat)
        if w >= 2:
            t = w - 2
            c = relu(y0[:, 0:32] + y1[:, 32:64] + y2[:, 64:96] + ab)
            if t >= 1:
                u = u + _dot(jnp.maximum(c_prev, c),
                             uwa_ref[(t - 1) * 32:t * 32, :])
            c_prev = c

    ev = evit_ref[...]
    m = [_dot(ev, vitk_ref[d]) for d in range(3)]
    vb = vb_ref[...]
    vmat = v_ref[...]
    c_prev = None
    for t in range(28):
        s = vb
        for d in range(3):
            col = vmat[:, t + d:t + d + 1]
            md = m[d]
            for j in range(3):
                s = s + (col == j).astype(jnp.float32) * md[j:j + 1, :]
        c = relu(s)
        if t >= 1:
            u = u + _dot(jnp.maximum(c_prev, c),
                         uwv_ref[(t - 1) * 32:t * 32, :])
        c_prev = c

    user_emb = relu(_dot(relu(u), uw2_ref[...]) + ub2_ref[...])

    p = pb1_ref[...]
    for i in range(5):
        p = p + _dot(gs_ref[i], pw1_ref[i])
    phone_emb = relu(_dot(relu(p), pw2_ref[...]) + pb2_ref[...])

    r = relu(_dot(user_emb, rw1_ref[0:32, :]) +
             _dot(phone_emb, rw1_ref[32:64, :]) + rb1_ref[...])
    out_ref[...] = jax.nn.sigmoid(_dot(r, rw2_ref[...]) + rb2_ref[...])


def _full(shape):
    nd = len(shape)
    return pl.BlockSpec(shape, lambda i, nd=nd: (0,) * nd)


def kernel(brand, modelname, version, phone_log_model, phone_raw_model,
           total_use_days, user_age, user_sex, user_degree, resident_province,
           resident_city, resident_city_type, sale_channel_1, sale_channel_2,
           vatality, all_app, emb_brand, emb_model, emb_version, emb_phone_log,
           emb_phone_raw, phone_W1, phone_b1, phone_W2, phone_b2, emb_age,
           emb_sex, emb_degree, emb_province, emb_city, emb_city_type, emb_ch1,
           emb_ch2, emb_use_days, emb_vitality, conv_vit_k, conv_vit_b,
           conv_app_k, conv_app_b, user_W1, user_b1, user_W2, user_b2,
           rate_W1, rate_b1, rate_W2, rate_b2):
    f32 = jnp.float32
    tables = [emb_brand, emb_model, emb_version, emb_phone_log, emb_phone_raw,
              emb_age, emb_sex, emb_degree, emb_province, emb_city,
              emb_city_type, emb_ch1, emb_ch2]
    stacked = jnp.pad(jnp.concatenate(tables, axis=0), ((0, 0), (0, 6)))
    offs, acc = [], 0
    for s in _SIZES:
        offs.append(acc)
        acc += s
    idx_cols = [brand, modelname, version, phone_log_model, phone_raw_model,
                user_age, user_sex, user_degree, resident_province,
                resident_city, resident_city_type, sale_channel_1,
                sale_channel_2]
    idx_all = jnp.concatenate(
        [c[:, 0].astype(jnp.int32) + o for c, o in zip(idx_cols, offs)])
    days_tab = jnp.pad(emb_use_days, ((0, 0), (0, 12)))
    days_idx = total_use_days[:, 0].astype(jnp.int32)

    g_small, g_days = _sc_gather(stacked, idx_all, days_tab, days_idx)

    pw1 = jnp.pad(phone_W1.reshape(5, 10, 128), ((0, 0), (0, 6), (0, 0)))
    uws = jnp.pad(user_W1[0:80].reshape(8, 10, 128), ((0, 0), (0, 6), (0, 0)))
    uwd = jnp.pad(user_W1[80:180], ((0, 12), (0, 0)))
    uwa = user_W1[180:1044]
    uwv = user_W1[1044:1908]
    kcat = jnp.concatenate([conv_app_k[0], conv_app_k[1], conv_app_k[2]],
                           axis=1)

    out = pl.pallas_call(
        _tc_body,
        grid=(B // BB,),
        in_specs=[
            pl.BlockSpec((_NT, BB, 16), lambda i: (0, i, 0)),
            pl.BlockSpec((BB, 112), lambda i: (i, 0)),
            pl.BlockSpec((BB, 30, 609), lambda i: (i, 0, 0)),
            pl.BlockSpec((BB, 30), lambda i: (i, 0)),
            _full((5, 16, 128)), _full((1, 128)), _full((128, 32)),
            _full((1, 32)),
            _full((8, 16, 128)), _full((112, 128)), _full((864, 128)),
            _full((864, 128)), _full((1, 128)), _full((128, 32)),
            _full((1, 32)),
            _full((609, 96)), _full((1, 32)), _full((3, 10)),
            _full((3, 10, 32)), _full((1, 32)),
            _full((64, 64)), _full((1, 64)), _full((64, 1)), _full((1, 1)),
        ],
        out_specs=pl.BlockSpec((BB, 1), lambda i: (i, 0)),
        out_shape=jax.ShapeDtypeStruct((B, 1), f32),
        compiler_params=pltpu.CompilerParams(
            dimension_semantics=("arbitrary",)),
    )(g_small, g_days, all_app, vatality,
      pw1, phone_b1.reshape(1, 128), phone_W2, phone_b2.reshape(1, 32),
      uws, uwd, uwa, uwv, user_b1.reshape(1, 128), user_W2,
      user_b2.reshape(1, 32),
      kcat, conv_app_b.reshape(1, 32), emb_vitality, conv_vit_k,
      conv_vit_b.reshape(1, 32),
      rate_W1, rate_b1.reshape(1, 64), rate_W2, rate_b2.reshape(1, 1))
    return out
